# baseline (speedup 1.0000x reference)
import functools

import jax
import jax.numpy as jnp
import numpy as np
from jax import lax
from jax.experimental import pallas as pl
from jax.experimental.pallas import tpu as pltpu
from jax.experimental.pallas import tpu_sc as plsc

_N = 10000
_E = 160000
_NFEAT = 128
_HDIM = 64
_NCLASS = 10
_NC = 2
_NS = 16
_NW = _NC * _NS
_CHUNK = 128
_NCHUNK = 40
_EPW = _CHUNK * _NCHUNK
_EPAD = _NW * _EPW
_ROWS_PER_SUB = 640
_NPAD = _NS * _ROWS_PER_SUB

_F32 = jnp.float32


def _kpair(j, i):
    return j * (j - 1) // 2 + i


_BROW = np.array([[j - 1] * 12 for j in range(1, 7) for i in range(j)])
_BCOL = np.array([[i * 12 + 1 + t for t in range(12)]
                  for j in range(1, 7) for i in range(j)])


def _mp_body(with_deg, u_hbm, src_hbm, dst_hbm, *rest):
    G = 1 if with_deg else 2
    if with_deg:
        (s_out, deg_out, src_v, dst_v, *bufs,
         ones16, zbuf16, u_sp, acc, acc16, gs0, gs1, ss0, ss1) = rest
    else:
        (s_out, src_v, dst_v, *bufs,
         u_sp, acc, gs0, gs1, ss0, ss1) = rest
    gsem = (gs0, gs1)
    ssem = (ss0, ss1)
    cid = lax.axis_index("c")
    sid = lax.axis_index("s")
    wid = sid * _NC + cid

    pltpu.sync_copy(src_hbm.at[wid], src_v)
    pltpu.sync_copy(dst_hbm.at[wid], dst_v)
    urows = _N // _NS
    pltpu.sync_copy(u_hbm.at[pl.ds(sid * urows, urows)],
                    u_sp.at[pl.ds(sid * urows, urows)])

    @pl.loop(0, _CHUNK)
    def _zfill(r):
        z16 = jnp.zeros((16,), _F32)
        for cc in range(_HDIM // 16):
            bufs[0][r, pl.ds(cc * 16, 16)] = z16
        if with_deg:
            zbuf16[r, pl.ds(0, 16)] = z16
            ones16[r, pl.ds(0, 16)] = jnp.ones((16,), _F32)

    for q in range(_ROWS_PER_SUB // _CHUNK):
        row0 = (sid * (_ROWS_PER_SUB // _CHUNK) + q) * _CHUNK
        pltpu.sync_copy(bufs[0], acc.at[pl.ds(row0, _CHUNK)])
        if with_deg:
            pltpu.sync_copy(zbuf16, acc16.at[pl.ds(row0, _CHUNK)])
    plsc.subcore_barrier()

    def _gather(c, b, g):
        pltpu.async_copy(u_sp.at[src_v.at[c]], bufs[b], gsem[g])

    def _gwait(b, g):
        pltpu.make_async_copy(u_sp.at[src_v.at[0]], bufs[b], gsem[g]).wait()

    def _scat(c, b, g):
        pltpu.async_copy(bufs[b], acc.at[dst_v.at[c]], ssem[g], add=True)
        if with_deg:
            pltpu.sync_copy(ones16, acc16.at[dst_v.at[c]], add=True)

    def _swait(b, g):
        pltpu.make_async_copy(bufs[b], acc.at[dst_v.at[0]], ssem[g]).wait()

    def _round(base, grp, issue_next):
        off = grp * G
        for m in range(G):
            _gwait(off + m, grp)
            _scat(base + m, off + m, grp)
        if issue_next:
            offn = (1 - grp) * G
            for m in range(G):
                _swait(offn + m, 1 - grp)
                _gather(base + G + m, offn + m, 1 - grp)

    nrounds = _NCHUNK // G
    for m in range(G):
        _gather(m, m, 0)
    for m in range(G):
        _gwait(m, 0)
        _scat(m, m, 0)
    for m in range(G):
        _gather(G + m, G + m, 1)

    @pl.loop(0, (nrounds - 2) // 2)
    def _pipe(p):
        b1 = (p * 2 + 1) * G
        _round(b1, 1, True)
        _round(b1 + G, 0, True)

    for m in range(G):
        _gwait(G + m, 1)
        _scat(_NCHUNK - G + m, G + m, 1)
    for m in range(G):
        _swait(m, 0)
        _swait(G + m, 1)
    plsc.subcore_barrier()

    row0 = sid * _ROWS_PER_SUB
    pltpu.sync_copy(acc.at[pl.ds(row0, _ROWS_PER_SUB)],
                    s_out.at[cid, pl.ds(row0, _ROWS_PER_SUB)])
    if with_deg:
        pltpu.sync_copy(acc16.at[pl.ds(row0, _ROWS_PER_SUB)],
                        deg_out.at[cid, pl.ds(row0, _ROWS_PER_SUB)])


@functools.lru_cache(maxsize=None)
def _make_mp(with_deg):
    mesh = plsc.VectorSubcoreMesh(core_axis_name="c", subcore_axis_name="s",
                                  num_cores=_NC, num_subcores=_NS)
    nbuf = 2 if with_deg else 4
    outs = [jax.ShapeDtypeStruct((_NC, _NPAD, _HDIM), _F32)]
    scratch = [
        pltpu.VMEM((_NCHUNK, _CHUNK), jnp.int32),
        pltpu.VMEM((_NCHUNK, _CHUNK), jnp.int32),
    ]
    scratch += [pltpu.VMEM((_CHUNK, _HDIM), _F32) for _ in range(nbuf)]
    if with_deg:
        outs.append(jax.ShapeDtypeStruct((_NC, _NPAD, 16), _F32))
        scratch += [
            pltpu.VMEM((_CHUNK, 16), _F32),
            pltpu.VMEM((_CHUNK, 16), _F32),
        ]
    scratch.append(pltpu.VMEM_SHARED((_N, _HDIM), _F32))
    scratch.append(pltpu.VMEM_SHARED((_NPAD, _HDIM), _F32))
    if with_deg:
        scratch.append(pltpu.VMEM_SHARED((_NPAD, 16), _F32))
    scratch += [pltpu.SemaphoreType.DMA] * 4
    return pl.kernel(
        functools.partial(_mp_body, with_deg),
        out_type=tuple(outs),
        mesh=mesh,
        scratch_types=scratch,
        compiler_params=pltpu.CompilerParams(use_tc_tiling_on_sc=False),
    )


def _run_mp_deg(u, src_r, dst_r):
    return _make_mp(True)(u, src_r, dst_r)


def _run_mp(u, src_r, dst_r):
    return _make_mp(False)(u, src_r, dst_r)


_BLK = 2000
_GRID = _N // _BLK

_VSPEC = pl.BlockSpec(memory_space=pltpu.MemorySpace.VMEM)
_SSPEC = pl.BlockSpec(memory_space=pltpu.MemorySpace.SMEM)


def _rows(shape_tail):
    return pl.BlockSpec((_BLK,) + shape_tail, lambda i: (i,) + (0,) * len(shape_tail))


def _part_rows(shape_tail):
    return pl.BlockSpec((2, _BLK) + shape_tail,
                        lambda i: (0, i) + (0,) * len(shape_tail))


def _const(shape):
    return pl.BlockSpec(shape, lambda i: (0,) * len(shape))


def _dot(m, w):
    return jnp.dot(m, w, precision=lax.Precision.HIGHEST,
                   preferred_element_type=_F32)


def _mix(h, a_ref):
    ex = jnp.exp(h - jnp.max(h, axis=1, keepdims=True))
    sm = ex / jnp.sum(ex, axis=1, keepdims=True)
    return (a_ref[0] * jax.nn.sigmoid(h) + a_ref[1] * jnp.tanh(h)
            + a_ref[2] * jax.nn.relu(h) + a_ref[3] * sm + a_ref[4] * h)


def _wcomb_body(wc_ref, bc_ref, wv_ref, weff_ref, beffp_ref):
    for k in range(21):
        wacc = wv_ref[k, 0] * wc_ref[12 * k]
        bacc = wv_ref[k, 0] * bc_ref[12 * k:12 * k + 1, :]
        for t in range(1, 12):
            wacc = wacc + wv_ref[k, t] * wc_ref[12 * k + t]
            bacc = bacc + wv_ref[k, t] * bc_ref[12 * k + t:12 * k + t + 1, :]
        weff_ref[pl.ds(64 * k, 64), :] = wacc
        beffp_ref[k:k + 1, :] = bacc


_wcomb = pl.pallas_call(
    _wcomb_body,
    out_shape=(
        jax.ShapeDtypeStruct((21 * _HDIM, _HDIM), _F32),
        jax.ShapeDtypeStruct((24, _HDIM), _F32),
    ),
    in_specs=[_VSPEC, _VSPEC, _SSPEC],
)


_ASPEC = pl.BlockSpec(memory_space=pl.ANY)


def _ys0_body(x_ref, wx_ref, bx_ref, weff_ref, a_ref, ys0_ref, u1_ref,
              u_scr, sem):
    i = pl.program_id(0)
    h0 = _dot(x_ref[...], wx_ref[...]) + bx_ref[...]
    xm = _mix(h0, a_ref)
    ys0_ref[...] = xm
    u_scr[...] = _dot(xm, weff_ref[pl.ds(0, _HDIM), :])
    pltpu.sync_copy(u_scr, u1_ref.at[pl.ds(i * _BLK, _BLK), :])


_ys0_call = pl.pallas_call(
    _ys0_body,
    grid=(_GRID,),
    out_shape=(
        jax.ShapeDtypeStruct((_N, _HDIM), _F32),
        jax.ShapeDtypeStruct((_N, _HDIM), _F32),
    ),
    in_specs=[
        _rows((_NFEAT,)),
        _const((_NFEAT, _HDIM)),
        _const((1, _HDIM)),
        _const((21 * _HDIM, _HDIM)),
        _SSPEC,
    ],
    out_specs=(_rows((_HDIM,)), _ASPEC),
    scratch_shapes=[pltpu.VMEM((_BLK, _HDIM), _F32), pltpu.SemaphoreType.DMA],
)


def _combine_body(j, sfull_ref, dinv_ref, beff_ref, weff_ref, xo_ref,
                  *ys_and_out):
    ys_refs = ys_and_out[:j]
    ysj_ref, unext_ref, xoj_ref = ys_and_out[j:j + 3]
    k = j + 3
    if j == 1:
        dinv_out = ys_and_out[k]
        k += 1
        s_scr, d_scr, u_scr, sem = ys_and_out[k:k + 4]
    else:
        s_scr, u_scr, sem = ys_and_out[k:k + 3]
    i = pl.program_id(0)

    pltpu.sync_copy(sfull_ref.at[:, pl.ds(i * _BLK, _BLK), :], s_scr)
    s = s_scr[0] + s_scr[1]
    if j == 1:
        pltpu.sync_copy(dinv_ref.at[:, pl.ds(i * _BLK, _BLK), :], d_scr)
        degs = d_scr[0, :, :1] + d_scr[1, :, :1]
        dinv = 1.0 / jnp.maximum(degs, 1.0)
        dinv_out[...] = dinv
    else:
        dinv = dinv_ref[...]
    beff = beff_ref[...]
    brow = jnp.zeros((1, _HDIM), _F32)
    for i2 in range(j):
        brow = brow + beff[_kpair(j, i2):_kpair(j, i2) + 1, :]
    ysj = dinv * s + brow
    ysj_ref[...] = ysj
    if j > 1:
        xoj_ref[...] = xo_ref[...] + ysj
    else:
        xoj_ref[...] = ysj
    cat = jnp.concatenate([ys_refs[i2][...] for i2 in range(j)] + [ysj], axis=1)
    wstk = weff_ref[pl.ds(64 * _kpair(j + 1, 0), 64 * (j + 1)), :]
    u_scr[...] = _dot(cat, wstk)
    pltpu.sync_copy(u_scr, unext_ref.at[pl.ds(i * _BLK, _BLK), :])


def _make_combine(j):
    out_shape = [
        jax.ShapeDtypeStruct((_N, _HDIM), _F32),
        jax.ShapeDtypeStruct((_N, _HDIM), _F32),
        jax.ShapeDtypeStruct((_N, _HDIM), _F32),
    ]
    out_specs = [_rows((_HDIM,)), _ASPEC, _rows((_HDIM,))]
    if j == 1:
        out_shape.append(jax.ShapeDtypeStruct((_N, 1), _F32))
        out_specs.append(_rows((1,)))
    in_specs = [
        _ASPEC,
        _ASPEC if j == 1 else _rows((1,)),
        _const((24, _HDIM)),
        _const((21 * _HDIM, _HDIM)),
        _rows((_HDIM,)),
    ] + [_rows((_HDIM,))] * j
    scratch = [pltpu.VMEM((2, _BLK, _HDIM), _F32)]
    if j == 1:
        scratch.append(pltpu.VMEM((2, _BLK, 16), _F32))
    scratch += [pltpu.VMEM((_BLK, _HDIM), _F32), pltpu.SemaphoreType.DMA]
    return pl.pallas_call(
        functools.partial(_combine_body, j),
        grid=(_GRID,),
        out_shape=tuple(out_shape),
        in_specs=in_specs,
        out_specs=tuple(out_specs),
        scratch_shapes=scratch,
    )


_combine = {j: _make_combine(j) for j in range(1, 6)}


def _final_body(sfull_ref, dinv_ref, beff_ref, xo_ref, wz_ref, bz_ref, g_ref,
                out_ref, s_scr, sem):
    i = pl.program_id(0)
    pltpu.sync_copy(sfull_ref.at[:, pl.ds(i * _BLK, _BLK), :], s_scr)
    s = s_scr[0] + s_scr[1]
    beff = beff_ref[...]
    brow = jnp.zeros((1, _HDIM), _F32)
    for i in range(6):
        brow = brow + beff[_kpair(6, i):_kpair(6, i) + 1, :]
    ys6 = dinv_ref[...] * s + brow
    xo = xo_ref[...] + ys6
    zh = _dot(xo, wz_ref[...]) + bz_ref[...]
    zh = zh[:, :_NCLASS]
    ex = jnp.exp(zh - jnp.max(zh, axis=1, keepdims=True))
    sm = ex / jnp.sum(ex, axis=1, keepdims=True)
    out_ref[...] = (g_ref[0] * jax.nn.sigmoid(zh) + g_ref[1] * jnp.tanh(zh)
                    + g_ref[2] * jax.nn.relu(zh) + g_ref[3] * sm
                    + g_ref[4] * zh)


_final = pl.pallas_call(
    _final_body,
    grid=(_GRID,),
    out_shape=jax.ShapeDtypeStruct((_N, _NCLASS), _F32),
    in_specs=[
        _ASPEC,
        _rows((1,)),
        _const((24, _HDIM)),
        _rows((_HDIM,)),
        _const((_HDIM, 128)),
        _const((1, 128)),
        _SSPEC,
    ],
    out_specs=_rows((_NCLASS,)),
    scratch_shapes=[pltpu.VMEM((2, _BLK, _HDIM), _F32),
                    pltpu.SemaphoreType.DMA],
)


def kernel(x, edge_index, W_x, b_x, Wc, bc, W_z, b_z, alpha, gamma, betas):
    a = jax.nn.softmax(alpha)
    g = jax.nn.softmax(gamma)
    wv = jax.nn.softmax(betas[_BROW, _BCOL], axis=1)

    pad = _EPAD - _E
    src = jnp.concatenate([edge_index[0], jnp.zeros((pad,), jnp.int32)])
    dst = jnp.concatenate([edge_index[1], jnp.full((pad,), _N, jnp.int32)])
    src_r = src.reshape(_NW, _NCHUNK, _CHUNK)
    dst_r = dst.reshape(_NW, _NCHUNK, _CHUNK)

    wz_pad = jnp.zeros((_HDIM, 128), _F32).at[:, :_NCLASS].set(W_z)
    bz_pad = jnp.zeros((1, 128), _F32).at[0, :_NCLASS].set(b_z)

    weff, beff = _wcomb(Wc, bc, wv)
    ys0, u1 = _ys0_call(x, W_x, b_x.reshape(1, _HDIM), weff, a)

    ys = [ys0]
    u = u1
    xo = None
    dinv = None
    for j in range(1, 7):
        if j == 1:
            sfull, degfull = _run_mp_deg(u, src_r, dst_r)
        else:
            (sfull,) = _run_mp(u, src_r, dst_r)
        if j < 6:
            if j == 1:
                ysj, u, xo, dinv = _combine[j](sfull, degfull, beff, weff,
                                               ys0, *ys)
            else:
                ysj, u, xo = _combine[j](sfull, dinv, beff, weff, xo, *ys)
            ys.append(ysj)
        else:
            out = _final(sfull, dinv, beff, xo, wz_pad, bz_pad, g)
    return out


# speedup vs baseline: 1.1807x; 1.1807x over previous
import functools

import jax
import jax.numpy as jnp
import numpy as np
from jax import lax
from jax.experimental import pallas as pl
from jax.experimental.pallas import tpu as pltpu
from jax.experimental.pallas import tpu_sc as plsc

_N = 10000
_E = 160000
_NFEAT = 128
_HDIM = 64
_NCLASS = 10
_NC = 2
_NS = 16
_NW = _NC * _NS
_CHUNK = 128
_NCHUNK = 40
_EPW = _CHUNK * _NCHUNK
_EPAD = _NW * _EPW
_ROWS_PER_SUB = 640
_NPAD = _NS * _ROWS_PER_SUB

_F32 = jnp.float32


def _kpair(j, i):
    return j * (j - 1) // 2 + i


_BROW = np.array([[j - 1] * 12 for j in range(1, 7) for i in range(j)])
_BCOL = np.array([[i * 12 + 1 + t for t in range(12)]
                  for j in range(1, 7) for i in range(j)])


def _mp_body(with_deg, u_hbm, src_hbm, dst_hbm, *rest):
    G = 1 if with_deg else 2
    if with_deg:
        (s_out, deg_out, src_v, dst_v, *bufs,
         ones16, zbuf16, u_sp, acc, acc16, gs0, gs1, ss0, ss1) = rest
    else:
        (s_out, src_v, dst_v, *bufs,
         u_sp, acc, gs0, gs1, ss0, ss1) = rest
    gsem = (gs0, gs1)
    ssem = (ss0, ss1)
    cid = lax.axis_index("c")
    sid = lax.axis_index("s")
    wid = sid * _NC + cid

    pltpu.sync_copy(src_hbm.at[wid], src_v)
    pltpu.sync_copy(dst_hbm.at[wid], dst_v)
    urows = _N // _NS
    pltpu.sync_copy(u_hbm.at[pl.ds(sid * urows, urows)],
                    u_sp.at[pl.ds(sid * urows, urows)])

    @pl.loop(0, _CHUNK)
    def _zfill(r):
        z16 = jnp.zeros((16,), _F32)
        for cc in range(_HDIM // 16):
            bufs[0][r, pl.ds(cc * 16, 16)] = z16
        if with_deg:
            zbuf16[r, pl.ds(0, 16)] = z16
            ones16[r, pl.ds(0, 16)] = jnp.ones((16,), _F32)

    for q in range(_ROWS_PER_SUB // _CHUNK):
        row0 = (sid * (_ROWS_PER_SUB // _CHUNK) + q) * _CHUNK
        pltpu.sync_copy(bufs[0], acc.at[pl.ds(row0, _CHUNK)])
        if with_deg:
            pltpu.sync_copy(zbuf16, acc16.at[pl.ds(row0, _CHUNK)])
    plsc.subcore_barrier()

    def _gather(c, b, g):
        pltpu.async_copy(u_sp.at[src_v.at[c]], bufs[b], gsem[g])

    def _gwait(b, g):
        pltpu.make_async_copy(u_sp.at[src_v.at[0]], bufs[b], gsem[g]).wait()

    def _scat(c, b, g):
        pltpu.async_copy(bufs[b], acc.at[dst_v.at[c]], ssem[g], add=True)
        if with_deg:
            pltpu.sync_copy(ones16, acc16.at[dst_v.at[c]], add=True)

    def _swait(b, g):
        pltpu.make_async_copy(bufs[b], acc.at[dst_v.at[0]], ssem[g]).wait()

    def _round(base, grp, issue_next):
        off = grp * G
        for m in range(G):
            _gwait(off + m, grp)
            _scat(base + m, off + m, grp)
        if issue_next:
            offn = (1 - grp) * G
            for m in range(G):
                _swait(offn + m, 1 - grp)
                _gather(base + G + m, offn + m, 1 - grp)

    nrounds = _NCHUNK // G
    for m in range(G):
        _gather(m, m, 0)
    for m in range(G):
        _gwait(m, 0)
        _scat(m, m, 0)
    for m in range(G):
        _gather(G + m, G + m, 1)

    @pl.loop(0, (nrounds - 2) // 2)
    def _pipe(p):
        b1 = (p * 2 + 1) * G
        _round(b1, 1, True)
        _round(b1 + G, 0, True)

    for m in range(G):
        _gwait(G + m, 1)
        _scat(_NCHUNK - G + m, G + m, 1)
    for m in range(G):
        _swait(m, 0)
        _swait(G + m, 1)
    plsc.subcore_barrier()

    row0 = sid * _ROWS_PER_SUB
    pltpu.sync_copy(acc.at[pl.ds(row0, _ROWS_PER_SUB)],
                    s_out.at[cid, pl.ds(row0, _ROWS_PER_SUB)])
    if with_deg:
        pltpu.sync_copy(acc16.at[pl.ds(row0, _ROWS_PER_SUB)],
                        deg_out.at[cid, pl.ds(row0, _ROWS_PER_SUB)])


@functools.lru_cache(maxsize=None)
def _make_mp(with_deg):
    mesh = plsc.VectorSubcoreMesh(core_axis_name="c", subcore_axis_name="s",
                                  num_cores=_NC, num_subcores=_NS)
    nbuf = 2 if with_deg else 4
    outs = [jax.ShapeDtypeStruct((_NC, _NPAD, _HDIM), _F32)]
    scratch = [
        pltpu.VMEM((_NCHUNK, _CHUNK), jnp.int32),
        pltpu.VMEM((_NCHUNK, _CHUNK), jnp.int32),
    ]
    scratch += [pltpu.VMEM((_CHUNK, _HDIM), _F32) for _ in range(nbuf)]
    if with_deg:
        outs.append(jax.ShapeDtypeStruct((_NC, _NPAD, 16), _F32))
        scratch += [
            pltpu.VMEM((_CHUNK, 16), _F32),
            pltpu.VMEM((_CHUNK, 16), _F32),
        ]
    scratch.append(pltpu.VMEM_SHARED((_N, _HDIM), _F32))
    scratch.append(pltpu.VMEM_SHARED((_NPAD, _HDIM), _F32))
    if with_deg:
        scratch.append(pltpu.VMEM_SHARED((_NPAD, 16), _F32))
    scratch += [pltpu.SemaphoreType.DMA] * 4
    return pl.kernel(
        functools.partial(_mp_body, with_deg),
        out_type=tuple(outs),
        mesh=mesh,
        scratch_types=scratch,
        compiler_params=pltpu.CompilerParams(use_tc_tiling_on_sc=False),
    )


def _run_mp_deg(u, src_r, dst_r):
    return _make_mp(True)(u, src_r, dst_r)


def _run_mp(u, src_r, dst_r):
    return _make_mp(False)(u, src_r, dst_r)


_BLK = 2000
_GRID = _N // _BLK

_VSPEC = pl.BlockSpec(memory_space=pltpu.MemorySpace.VMEM)
_SSPEC = pl.BlockSpec(memory_space=pltpu.MemorySpace.SMEM)


def _rows(shape_tail):
    return pl.BlockSpec((_BLK,) + shape_tail, lambda i: (i,) + (0,) * len(shape_tail))


def _part_rows(shape_tail):
    return pl.BlockSpec((2, _BLK) + shape_tail,
                        lambda i: (0, i) + (0,) * len(shape_tail))


def _const(shape):
    return pl.BlockSpec(shape, lambda i: (0,) * len(shape))


def _dot(m, w):
    return jnp.dot(m, w, precision=lax.Precision.HIGHEST,
                   preferred_element_type=_F32)


def _mix(h, a_ref):
    ex = jnp.exp(h - jnp.max(h, axis=1, keepdims=True))
    sm = ex / jnp.sum(ex, axis=1, keepdims=True)
    return (a_ref[0] * jax.nn.sigmoid(h) + a_ref[1] * jnp.tanh(h)
            + a_ref[2] * jax.nn.relu(h) + a_ref[3] * sm + a_ref[4] * h)


def _wcomb_body(wc_ref, bc_ref, wv_ref, weff_ref, beffp_ref):
    for k in range(21):
        wacc = wv_ref[k, 0] * wc_ref[12 * k]
        bacc = wv_ref[k, 0] * bc_ref[12 * k:12 * k + 1, :]
        for t in range(1, 12):
            wacc = wacc + wv_ref[k, t] * wc_ref[12 * k + t]
            bacc = bacc + wv_ref[k, t] * bc_ref[12 * k + t:12 * k + t + 1, :]
        weff_ref[pl.ds(64 * k, 64), :] = wacc
        beffp_ref[k:k + 1, :] = bacc


_wcomb = pl.pallas_call(
    _wcomb_body,
    out_shape=(
        jax.ShapeDtypeStruct((21 * _HDIM, _HDIM), _F32),
        jax.ShapeDtypeStruct((24, _HDIM), _F32),
    ),
    in_specs=[_VSPEC, _VSPEC, _SSPEC],
)


_ASPEC = pl.BlockSpec(memory_space=pl.ANY)


def _ys0_body(x_ref, wx_ref, bx_ref, weff_ref, a_ref, ys0_ref, u1_ref):
    h0 = _dot(x_ref[...], wx_ref[...]) + bx_ref[...]
    xm = _mix(h0, a_ref)
    ys0_ref[...] = xm
    u1_ref[...] = _dot(xm, weff_ref[pl.ds(0, _HDIM), :])


_ys0_call = pl.pallas_call(
    _ys0_body,
    grid=(_GRID,),
    out_shape=(
        jax.ShapeDtypeStruct((_N, _HDIM), _F32),
        jax.ShapeDtypeStruct((_N, _HDIM), _F32),
    ),
    in_specs=[
        _rows((_NFEAT,)),
        _const((_NFEAT, _HDIM)),
        _const((1, _HDIM)),
        _const((21 * _HDIM, _HDIM)),
        _SSPEC,
    ],
    out_specs=(_rows((_HDIM,)), _rows((_HDIM,))),
)


def _combine_body(j, sfull_ref, dinv_ref, beff_ref, weff_ref, xo_ref,
                  *ys_and_out):
    ys_refs = ys_and_out[:j]
    ysj_ref, unext_ref, xoj_ref = ys_and_out[j:j + 3]
    dinv_out = ys_and_out[j + 3] if j == 1 else None

    s = sfull_ref[0] + sfull_ref[1]
    if j == 1:
        degs = dinv_ref[0, :, :1] + dinv_ref[1, :, :1]
        dinv = 1.0 / jnp.maximum(degs, 1.0)
        dinv_out[...] = dinv
    else:
        dinv = dinv_ref[...]
    beff = beff_ref[...]
    brow = jnp.zeros((1, _HDIM), _F32)
    for i2 in range(j):
        brow = brow + beff[_kpair(j, i2):_kpair(j, i2) + 1, :]
    ysj = dinv * s + brow
    ysj_ref[...] = ysj
    if j > 1:
        xoj_ref[...] = xo_ref[...] + ysj
    else:
        xoj_ref[...] = ysj
    cat = jnp.concatenate([ys_refs[i2][...] for i2 in range(j)] + [ysj], axis=1)
    wstk = weff_ref[pl.ds(64 * _kpair(j + 1, 0), 64 * (j + 1)), :]
    unext_ref[...] = _dot(cat, wstk)


def _make_combine(j):
    out_shape = [
        jax.ShapeDtypeStruct((_N, _HDIM), _F32),
        jax.ShapeDtypeStruct((_N, _HDIM), _F32),
        jax.ShapeDtypeStruct((_N, _HDIM), _F32),
    ]
    out_specs = [_rows((_HDIM,)), _rows((_HDIM,)), _rows((_HDIM,))]
    if j == 1:
        out_shape.append(jax.ShapeDtypeStruct((_N, 1), _F32))
        out_specs.append(_rows((1,)))
    in_specs = [
        _part_rows((_HDIM,)),
        _part_rows((16,)) if j == 1 else _rows((1,)),
        _const((24, _HDIM)),
        _const((21 * _HDIM, _HDIM)),
        _rows((_HDIM,)),
    ] + [_rows((_HDIM,))] * j
    return pl.pallas_call(
        functools.partial(_combine_body, j),
        grid=(_GRID,),
        out_shape=tuple(out_shape),
        in_specs=in_specs,
        out_specs=tuple(out_specs),
    )


_combine = {j: _make_combine(j) for j in range(1, 6)}


def _final_body(sfull_ref, dinv_ref, beff_ref, xo_ref, wz_ref, bz_ref, g_ref,
                out_ref):
    s = sfull_ref[0] + sfull_ref[1]
    beff = beff_ref[...]
    brow = jnp.zeros((1, _HDIM), _F32)
    for i in range(6):
        brow = brow + beff[_kpair(6, i):_kpair(6, i) + 1, :]
    ys6 = dinv_ref[...] * s + brow
    xo = xo_ref[...] + ys6
    zh = _dot(xo, wz_ref[...]) + bz_ref[...]
    zh = zh[:, :_NCLASS]
    ex = jnp.exp(zh - jnp.max(zh, axis=1, keepdims=True))
    sm = ex / jnp.sum(ex, axis=1, keepdims=True)
    out_ref[...] = (g_ref[0] * jax.nn.sigmoid(zh) + g_ref[1] * jnp.tanh(zh)
                    + g_ref[2] * jax.nn.relu(zh) + g_ref[3] * sm
                    + g_ref[4] * zh)


_final = pl.pallas_call(
    _final_body,
    grid=(_GRID,),
    out_shape=jax.ShapeDtypeStruct((_N, _NCLASS), _F32),
    in_specs=[
        _part_rows((_HDIM,)),
        _rows((1,)),
        _const((24, _HDIM)),
        _rows((_HDIM,)),
        _const((_HDIM, 128)),
        _const((1, 128)),
        _SSPEC,
    ],
    out_specs=_rows((_NCLASS,)),
)


def kernel(x, edge_index, W_x, b_x, Wc, bc, W_z, b_z, alpha, gamma, betas):
    a = jax.nn.softmax(alpha)
    g = jax.nn.softmax(gamma)
    wv = jax.nn.softmax(betas[_BROW, _BCOL], axis=1)

    pad = _EPAD - _E
    src = jnp.concatenate([edge_index[0], jnp.zeros((pad,), jnp.int32)])
    dst = jnp.concatenate([edge_index[1], jnp.full((pad,), _N, jnp.int32)])
    src_r = src.reshape(_NW, _NCHUNK, _CHUNK)
    dst_r = dst.reshape(_NW, _NCHUNK, _CHUNK)

    wz_pad = jnp.zeros((_HDIM, 128), _F32).at[:, :_NCLASS].set(W_z)
    bz_pad = jnp.zeros((1, 128), _F32).at[0, :_NCLASS].set(b_z)

    weff, beff = _wcomb(Wc, bc, wv)
    ys0, u1 = _ys0_call(x, W_x, b_x.reshape(1, _HDIM), weff, a)

    ys = [ys0]
    u = u1
    xo = None
    dinv = None
    for j in range(1, 7):
        if j == 1:
            sfull, degfull = _run_mp_deg(u, src_r, dst_r)
        else:
            (sfull,) = _run_mp(u, src_r, dst_r)
        if j < 6:
            if j == 1:
                ysj, u, xo, dinv = _combine[j](sfull, degfull, beff, weff,
                                               ys0, *ys)
            else:
                ysj, u, xo = _combine[j](sfull, dinv, beff, weff, xo, *ys)
            ys.append(ysj)
        else:
            out = _final(sfull, dinv, beff, xo, wz_pad, bz_pad, g)
    return out


# speedup vs baseline: 1.3882x; 1.1757x over previous
import functools

import jax
import jax.numpy as jnp
import numpy as np
from jax import lax
from jax.experimental import pallas as pl
from jax.experimental.pallas import tpu as pltpu
from jax.experimental.pallas import tpu_sc as plsc

_N = 10000
_E = 160000
_NFEAT = 128
_HDIM = 64
_NCLASS = 10
_NC = 2
_NS = 16
_NW = _NC * _NS
_CHUNK = 128
_NCHUNK = 40
_EPW = _CHUNK * _NCHUNK
_EPAD = _NW * _EPW
_ROWS_PER_SUB = 640
_NPAD = _NS * _ROWS_PER_SUB

_F32 = jnp.float32


def _kpair(j, i):
    return j * (j - 1) // 2 + i


_BROW = np.array([[j - 1] * 12 for j in range(1, 7) for i in range(j)])
_BCOL = np.array([[i * 12 + 1 + t for t in range(12)]
                  for j in range(1, 7) for i in range(j)])


def _mp_body(with_deg, u_hbm, src_hbm, dst_hbm, *rest):
    G = 1 if with_deg else 2
    if with_deg:
        (s_out, deg_out, src_v, dst_v, *bufs,
         ones16, zbuf16, u_sp, acc, acc16, gs0, gs1, ss0, ss1) = rest
    else:
        (s_out, src_v, dst_v, *bufs,
         u_sp, acc, gs0, gs1, ss0, ss1) = rest
    gsem = (gs0, gs1)
    ssem = (ss0, ss1)
    cid = lax.axis_index("c")
    sid = lax.axis_index("s")
    wid = sid * _NC + cid

    pltpu.sync_copy(src_hbm.at[wid], src_v)
    pltpu.sync_copy(dst_hbm.at[wid], dst_v)
    urows = _N // _NS
    pltpu.sync_copy(u_hbm.at[pl.ds(sid * urows, urows), pl.ds(0, _HDIM)],
                    u_sp.at[pl.ds(sid * urows, urows)])

    @pl.loop(0, _CHUNK)
    def _zfill(r):
        z16 = jnp.zeros((16,), _F32)
        for cc in range(_HDIM // 16):
            bufs[0][r, pl.ds(cc * 16, 16)] = z16
        if with_deg:
            zbuf16[r, pl.ds(0, 16)] = z16
            ones16[r, pl.ds(0, 16)] = jnp.ones((16,), _F32)

    for q in range(_ROWS_PER_SUB // _CHUNK):
        row0 = (sid * (_ROWS_PER_SUB // _CHUNK) + q) * _CHUNK
        pltpu.sync_copy(bufs[0], acc.at[pl.ds(row0, _CHUNK)])
        if with_deg:
            pltpu.sync_copy(zbuf16, acc16.at[pl.ds(row0, _CHUNK)])
    plsc.subcore_barrier()

    def _gather(c, b, g):
        pltpu.async_copy(u_sp.at[src_v.at[c]], bufs[b], gsem[g])

    def _gwait(b, g):
        pltpu.make_async_copy(u_sp.at[src_v.at[0]], bufs[b], gsem[g]).wait()

    def _scat(c, b, g):
        pltpu.async_copy(bufs[b], acc.at[dst_v.at[c]], ssem[g], add=True)
        if with_deg:
            pltpu.sync_copy(ones16, acc16.at[dst_v.at[c]], add=True)

    def _swait(b, g):
        pltpu.make_async_copy(bufs[b], acc.at[dst_v.at[0]], ssem[g]).wait()

    def _round(base, grp, issue_next):
        off = grp * G
        for m in range(G):
            _gwait(off + m, grp)
            _scat(base + m, off + m, grp)
        if issue_next:
            offn = (1 - grp) * G
            for m in range(G):
                _swait(offn + m, 1 - grp)
                _gather(base + G + m, offn + m, 1 - grp)

    nrounds = _NCHUNK // G
    for m in range(G):
        _gather(m, m, 0)
    for m in range(G):
        _gwait(m, 0)
        _scat(m, m, 0)
    for m in range(G):
        _gather(G + m, G + m, 1)

    @pl.loop(0, (nrounds - 2) // 2)
    def _pipe(p):
        b1 = (p * 2 + 1) * G
        _round(b1, 1, True)
        _round(b1 + G, 0, True)

    for m in range(G):
        _gwait(G + m, 1)
        _scat(_NCHUNK - G + m, G + m, 1)
    for m in range(G):
        _swait(m, 0)
        _swait(G + m, 1)
    plsc.subcore_barrier()

    row0 = sid * _ROWS_PER_SUB
    pltpu.sync_copy(acc.at[pl.ds(row0, _ROWS_PER_SUB)],
                    s_out.at[cid, pl.ds(row0, _ROWS_PER_SUB), pl.ds(0, _HDIM)])
    if with_deg:
        pltpu.sync_copy(acc16.at[pl.ds(row0, _ROWS_PER_SUB)],
                        deg_out.at[cid, pl.ds(row0, _ROWS_PER_SUB), pl.ds(0, 16)])


@functools.lru_cache(maxsize=None)
def _make_mp(with_deg):
    mesh = plsc.VectorSubcoreMesh(core_axis_name="c", subcore_axis_name="s",
                                  num_cores=_NC, num_subcores=_NS)
    nbuf = 2 if with_deg else 4
    outs = [jax.ShapeDtypeStruct((_NC, _NPAD, 128), _F32)]
    scratch = [
        pltpu.VMEM((_NCHUNK, _CHUNK), jnp.int32),
        pltpu.VMEM((_NCHUNK, _CHUNK), jnp.int32),
    ]
    scratch += [pltpu.VMEM((_CHUNK, _HDIM), _F32) for _ in range(nbuf)]
    if with_deg:
        outs.append(jax.ShapeDtypeStruct((_NC, _NPAD, 128), _F32))
        scratch += [
            pltpu.VMEM((_CHUNK, 16), _F32),
            pltpu.VMEM((_CHUNK, 16), _F32),
        ]
    scratch.append(pltpu.VMEM_SHARED((_N, _HDIM), _F32))
    scratch.append(pltpu.VMEM_SHARED((_NPAD, _HDIM), _F32))
    if with_deg:
        scratch.append(pltpu.VMEM_SHARED((_NPAD, 16), _F32))
    scratch += [pltpu.SemaphoreType.DMA] * 4
    return pl.kernel(
        functools.partial(_mp_body, with_deg),
        out_type=tuple(outs),
        mesh=mesh,
        scratch_types=scratch,
        compiler_params=pltpu.CompilerParams(use_tc_tiling_on_sc=False),
    )


def _run_mp_deg(u, src_r, dst_r):
    return _make_mp(True)(u, src_r, dst_r)


def _run_mp(u, src_r, dst_r):
    return _make_mp(False)(u, src_r, dst_r)


_BLK = 2000
_GRID = _N // _BLK

_VSPEC = pl.BlockSpec(memory_space=pltpu.MemorySpace.VMEM)
_SSPEC = pl.BlockSpec(memory_space=pltpu.MemorySpace.SMEM)


def _rows(shape_tail):
    return pl.BlockSpec((_BLK,) + shape_tail, lambda i: (i,) + (0,) * len(shape_tail))


def _part_rows(shape_tail):
    return pl.BlockSpec((2, _BLK) + shape_tail,
                        lambda i: (0, i) + (0,) * len(shape_tail))


def _const(shape):
    return pl.BlockSpec(shape, lambda i: (0,) * len(shape))


def _dot(m, w):
    return jnp.dot(m, w, precision=lax.Precision.HIGHEST,
                   preferred_element_type=_F32)


def _mix(h, a_ref):
    ex = jnp.exp(h - jnp.max(h, axis=1, keepdims=True))
    sm = ex / jnp.sum(ex, axis=1, keepdims=True)
    return (a_ref[0] * jax.nn.sigmoid(h) + a_ref[1] * jnp.tanh(h)
            + a_ref[2] * jax.nn.relu(h) + a_ref[3] * sm + a_ref[4] * h)


def _wcomb_body(wc_ref, bc_ref, wv_ref, weff_ref, beffp_ref):
    for k in range(21):
        wacc = wv_ref[k, 0] * wc_ref[12 * k]
        bacc = wv_ref[k, 0] * bc_ref[12 * k:12 * k + 1, :]
        for t in range(1, 12):
            wacc = wacc + wv_ref[k, t] * wc_ref[12 * k + t]
            bacc = bacc + wv_ref[k, t] * bc_ref[12 * k + t:12 * k + t + 1, :]
        weff_ref[pl.ds(64 * k, 64), :] = wacc
        beffp_ref[k:k + 1, :] = bacc


_wcomb = pl.pallas_call(
    _wcomb_body,
    out_shape=(
        jax.ShapeDtypeStruct((21 * _HDIM, _HDIM), _F32),
        jax.ShapeDtypeStruct((24, _HDIM), _F32),
    ),
    in_specs=[_VSPEC, _VSPEC, _SSPEC],
)


_ASPEC = pl.BlockSpec(memory_space=pl.ANY)


def _ys0_body(x_ref, wx_ref, bx_ref, weff_ref, a_ref, ys0_ref, u1_ref):
    h0 = _dot(x_ref[...], wx_ref[...]) + bx_ref[...]
    xm = _mix(h0, a_ref)
    ys0_ref[...] = xm
    un = _dot(xm, weff_ref[pl.ds(0, _HDIM), :])
    u1_ref[...] = jnp.concatenate([un, jnp.zeros_like(un)], axis=1)


_ys0_call = pl.pallas_call(
    _ys0_body,
    grid=(_GRID,),
    out_shape=(
        jax.ShapeDtypeStruct((_N, _HDIM), _F32),
        jax.ShapeDtypeStruct((_N, 128), _F32),
    ),
    in_specs=[
        _rows((_NFEAT,)),
        _const((_NFEAT, _HDIM)),
        _const((1, _HDIM)),
        _const((21 * _HDIM, _HDIM)),
        _SSPEC,
    ],
    out_specs=(_rows((_HDIM,)), _rows((128,))),
)


def _combine_body(j, sfull_ref, dinv_ref, beff_ref, weff_ref, xo_ref,
                  *ys_and_out):
    ys_refs = ys_and_out[:j]
    ysj_ref, unext_ref, xoj_ref = ys_and_out[j:j + 3]
    dinv_out = ys_and_out[j + 3] if j == 1 else None

    s = sfull_ref[0, :, :_HDIM] + sfull_ref[1, :, :_HDIM]
    if j == 1:
        degs = dinv_ref[0, :, :1] + dinv_ref[1, :, :1]
        dinv = 1.0 / jnp.maximum(degs, 1.0)
        dinv_out[...] = dinv
    else:
        dinv = dinv_ref[...]
    beff = beff_ref[...]
    brow = jnp.zeros((1, _HDIM), _F32)
    for i2 in range(j):
        brow = brow + beff[_kpair(j, i2):_kpair(j, i2) + 1, :]
    ysj = dinv * s + brow
    ysj_ref[...] = ysj
    if j > 1:
        xoj_ref[...] = xo_ref[...] + ysj
    else:
        xoj_ref[...] = ysj
    cat = jnp.concatenate([ys_refs[i2][...] for i2 in range(j)] + [ysj], axis=1)
    wstk = weff_ref[pl.ds(64 * _kpair(j + 1, 0), 64 * (j + 1)), :]
    un = _dot(cat, wstk)
    unext_ref[...] = jnp.concatenate([un, jnp.zeros_like(un)], axis=1)


def _make_combine(j):
    out_shape = [
        jax.ShapeDtypeStruct((_N, _HDIM), _F32),
        jax.ShapeDtypeStruct((_N, 128), _F32),
        jax.ShapeDtypeStruct((_N, _HDIM), _F32),
    ]
    out_specs = [_rows((_HDIM,)), _rows((128,)), _rows((_HDIM,))]
    if j == 1:
        out_shape.append(jax.ShapeDtypeStruct((_N, 1), _F32))
        out_specs.append(_rows((1,)))
    in_specs = [
        _part_rows((128,)),
        _part_rows((128,)) if j == 1 else _rows((1,)),
        _const((24, _HDIM)),
        _const((21 * _HDIM, _HDIM)),
        _rows((_HDIM,)),
    ] + [_rows((_HDIM,))] * j
    return pl.pallas_call(
        functools.partial(_combine_body, j),
        grid=(_GRID,),
        out_shape=tuple(out_shape),
        in_specs=in_specs,
        out_specs=tuple(out_specs),
    )


_combine = {j: _make_combine(j) for j in range(1, 6)}


def _final_body(sfull_ref, dinv_ref, beff_ref, xo_ref, wz_ref, bz_ref, g_ref,
                out_ref):
    s = sfull_ref[0, :, :_HDIM] + sfull_ref[1, :, :_HDIM]
    beff = beff_ref[...]
    brow = jnp.zeros((1, _HDIM), _F32)
    for i in range(6):
        brow = brow + beff[_kpair(6, i):_kpair(6, i) + 1, :]
    ys6 = dinv_ref[...] * s + brow
    xo = xo_ref[...] + ys6
    zh = _dot(xo, wz_ref[...]) + bz_ref[...]
    zh = zh[:, :_NCLASS]
    ex = jnp.exp(zh - jnp.max(zh, axis=1, keepdims=True))
    sm = ex / jnp.sum(ex, axis=1, keepdims=True)
    out_ref[...] = (g_ref[0] * jax.nn.sigmoid(zh) + g_ref[1] * jnp.tanh(zh)
                    + g_ref[2] * jax.nn.relu(zh) + g_ref[3] * sm
                    + g_ref[4] * zh)


_final = pl.pallas_call(
    _final_body,
    grid=(_GRID,),
    out_shape=jax.ShapeDtypeStruct((_N, _NCLASS), _F32),
    in_specs=[
        _part_rows((128,)),
        _rows((1,)),
        _const((24, _HDIM)),
        _rows((_HDIM,)),
        _const((_HDIM, 128)),
        _const((1, 128)),
        _SSPEC,
    ],
    out_specs=_rows((_NCLASS,)),
)


def kernel(x, edge_index, W_x, b_x, Wc, bc, W_z, b_z, alpha, gamma, betas):
    a = jax.nn.softmax(alpha)
    g = jax.nn.softmax(gamma)
    wv = jax.nn.softmax(betas[_BROW, _BCOL], axis=1)

    pad = _EPAD - _E
    src = jnp.concatenate([edge_index[0], jnp.zeros((pad,), jnp.int32)])
    dst = jnp.concatenate([edge_index[1], jnp.full((pad,), _N, jnp.int32)])
    src_r = src.reshape(_NW, _NCHUNK, _CHUNK)
    dst_r = dst.reshape(_NW, _NCHUNK, _CHUNK)

    wz_pad = jnp.zeros((_HDIM, 128), _F32).at[:, :_NCLASS].set(W_z)
    bz_pad = jnp.zeros((1, 128), _F32).at[0, :_NCLASS].set(b_z)

    weff, beff = _wcomb(Wc, bc, wv)
    ys0, u1 = _ys0_call(x, W_x, b_x.reshape(1, _HDIM), weff, a)

    ys = [ys0]
    u = u1
    xo = None
    dinv = None
    for j in range(1, 7):
        if j == 1:
            sfull, degfull = _run_mp_deg(u, src_r, dst_r)
        else:
            (sfull,) = _run_mp(u, src_r, dst_r)
        if j < 6:
            if j == 1:
                ysj, u, xo, dinv = _combine[j](sfull, degfull, beff, weff,
                                               ys0, *ys)
            else:
                ysj, u, xo = _combine[j](sfull, dinv, beff, weff, xo, *ys)
            ys.append(ysj)
        else:
            out = _final(sfull, dinv, beff, xo, wz_pad, bz_pad, g)
    return out
